# Initial kernel scaffold; baseline (speedup 1.0000x reference)
import functools

import jax
import jax.numpy as jnp
from jax.experimental import pallas as pl

ALPHA = 1.0
NN_SIZE = 5
RADIUS = 0.07
H = 0.03
EPS = 1e-12

B = 4
N = 2048
D = 3
TILE = 512


def _loss_kernel(pred_ref, predt_ref, gtt_ref, emd_ref, rep_ref):
    b = pl.program_id(0)
    t = pl.program_id(1)

    p = pred_ref[0]
    pt = predt_ref[0]
    gtt = gtt_ref[0]

    iota = jax.lax.broadcasted_iota(jnp.int32, (TILE, N), 1)
    a2 = jnp.sum(p * p, axis=1, keepdims=True)

    def gather_dist2(onehot_f32, coords_t):
        d2 = jnp.zeros((TILE, 1), dtype=jnp.float32)
        for c in range(D):
            sel = jnp.sum(onehot_f32 * coords_t[c:c + 1, :], axis=1,
                          keepdims=True)
            diff = p[:, c:c + 1] - sel
            d2 = d2 + diff * diff
        return d2

    g2 = jnp.sum(gtt * gtt, axis=0, keepdims=True)
    mm = jax.lax.dot_general(p, gtt, (((1,), (0,)), ((), ())),
                             preferred_element_type=jnp.float32)
    d = a2 + g2 - 2.0 * mm
    m = jnp.min(d, axis=1, keepdims=True)
    ii = jnp.where(d == m, iota, N)
    idx = jnp.min(ii, axis=1, keepdims=True)
    onehot = (ii == idx).astype(jnp.float32)
    emd_val = jnp.sum(gather_dist2(onehot, gtt))

    p2 = jnp.sum(pt * pt, axis=0, keepdims=True)
    mm2 = jax.lax.dot_general(p, pt, (((1,), (0,)), ((), ())),
                              preferred_element_type=jnp.float32)
    dq = a2 + p2 - 2.0 * mm2
    rep_val = jnp.zeros((), dtype=jnp.float32)
    for k in range(NN_SIZE):
        mk = jnp.min(dq, axis=1, keepdims=True)
        iik = jnp.where(dq == mk, iota, N)
        idxk = jnp.min(iik, axis=1, keepdims=True)
        oh = iik == idxk
        if k > 0:
            d2 = gather_dist2(oh.astype(jnp.float32), pt)
            d2 = jnp.maximum(d2, EPS)
            dist = jnp.sqrt(d2)
            w = jnp.exp(-d2 / (H * H))
            rep_val = rep_val + jnp.sum((RADIUS - dist) * w)
        dq = jnp.where(oh, jnp.inf, dq)

    @pl.when(t == 0)
    def _():
        emd_ref[0, 0] = emd_val

    @pl.when(t != 0)
    def _():
        emd_ref[0, 0] += emd_val

    @pl.when(jnp.logical_and(b == 0, t == 0))
    def _():
        rep_ref[0, 0] = rep_val

    @pl.when(jnp.logical_not(jnp.logical_and(b == 0, t == 0)))
    def _():
        rep_ref[0, 0] += rep_val


@functools.partial(jax.jit, static_argnames=("interpret",))
def kernel(pred, gt, pcd_radius, interpret=False):
    pred_t = jnp.transpose(pred, (0, 2, 1))
    gt_t = jnp.transpose(gt, (0, 2, 1))

    emd_sums, rep_sum = pl.pallas_call(
        _loss_kernel,
        grid=(B, N // TILE),
        in_specs=[
            pl.BlockSpec((1, TILE, D), lambda b, t: (b, t, 0)),
            pl.BlockSpec((1, D, N), lambda b, t: (b, 0, 0)),
            pl.BlockSpec((1, D, N), lambda b, t: (b, 0, 0)),
        ],
        out_specs=[
            pl.BlockSpec((1, 1), lambda b, t: (b, 0)),
            pl.BlockSpec((1, 1), lambda b, t: (0, 0)),
        ],
        out_shape=[
            jax.ShapeDtypeStruct((B, 1), jnp.float32),
            jax.ShapeDtypeStruct((1, 1), jnp.float32),
        ],
        interpret=interpret,
    )(pred, pred_t, gt_t)

    dist2_mean = emd_sums / float(N * D) / pcd_radius
    emd_loss = jnp.mean(dist2_mean) * 100.0
    uniform_loss = rep_sum[0, 0] / float(B * N * (NN_SIZE - 1))
    return (emd_loss, ALPHA * uniform_loss)


# speedup vs baseline: 15.0863x; 15.0863x over previous
import functools

import jax
import jax.numpy as jnp
from jax.experimental import pallas as pl

ALPHA = 1.0
NN_SIZE = 5
RADIUS = 0.07
H = 0.03
EPS = 1e-12

B = 4
N = 2048
D = 3
TILE = 512


def _loss_kernel(pred_ref, predt_ref, gtt_ref, emd_ref, rep_ref):
    b = pl.program_id(0)
    t = pl.program_id(1)

    p = pred_ref[0]
    pt = predt_ref[0]
    gtt = gtt_ref[0]

    iota = jax.lax.broadcasted_iota(jnp.int32, (TILE, N), 1)
    a2 = jnp.sum(p * p, axis=1, keepdims=True)

    def gather_dist2(onehot_f32, coords_t):
        d2 = jnp.zeros((TILE, 1), dtype=jnp.float32)
        for c in range(D):
            sel = jnp.sum(onehot_f32 * coords_t[c:c + 1, :], axis=1,
                          keepdims=True)
            diff = p[:, c:c + 1] - sel
            d2 = d2 + diff * diff
        return d2

    g2 = jnp.sum(gtt * gtt, axis=0, keepdims=True)
    mm = jax.lax.dot_general(p, gtt, (((1,), (0,)), ((), ())),
                             preferred_element_type=jnp.float32)
    d = a2 + g2 - 2.0 * mm
    m = jnp.min(d, axis=1, keepdims=True)
    ii = jnp.where(d == m, iota, N)
    idx = jnp.min(ii, axis=1, keepdims=True)
    onehot = (ii == idx).astype(jnp.float32)
    emd_val = jnp.sum(gather_dist2(onehot, gtt))

    p2 = jnp.sum(pt * pt, axis=0, keepdims=True)
    mm2 = jax.lax.dot_general(p, pt, (((1,), (0,)), ((), ())),
                              preferred_element_type=jnp.float32)
    dq = a2 + p2 - 2.0 * mm2
    rep_val = jnp.zeros((), dtype=jnp.float32)
    for k in range(NN_SIZE):
        mk = jnp.min(dq, axis=1, keepdims=True)
        iik = jnp.where(dq == mk, iota, N)
        idxk = jnp.min(iik, axis=1, keepdims=True)
        oh = iik == idxk
        if k > 0:
            d2 = gather_dist2(oh.astype(jnp.float32), pt)
            d2 = jnp.maximum(d2, EPS)
            dist = jnp.sqrt(d2)
            w = jnp.exp(-d2 / (H * H))
            rep_val = rep_val + jnp.sum((RADIUS - dist) * w)
        dq = jnp.where(oh, jnp.inf, dq)

    @pl.when(jnp.logical_and(b == 0, t == 0))
    def _():
        emd_ref[...] = jnp.zeros_like(emd_ref)
        rep_ref[...] = jnp.zeros_like(rep_ref)

    emd_ref[pl.ds(b, 1), :] += emd_val.reshape(1, 1)
    rep_ref[...] += rep_val.reshape(1, 1)


@functools.partial(jax.jit, static_argnames=("interpret",))
def kernel(pred, gt, pcd_radius, interpret=False):
    pred_t = jnp.transpose(pred, (0, 2, 1))
    gt_t = jnp.transpose(gt, (0, 2, 1))

    emd_sums, rep_sum = pl.pallas_call(
        _loss_kernel,
        grid=(B, N // TILE),
        in_specs=[
            pl.BlockSpec((1, TILE, D), lambda b, t: (b, t, 0)),
            pl.BlockSpec((1, D, N), lambda b, t: (b, 0, 0)),
            pl.BlockSpec((1, D, N), lambda b, t: (b, 0, 0)),
        ],
        out_specs=[
            pl.BlockSpec((B, 1), lambda b, t: (0, 0)),
            pl.BlockSpec((1, 1), lambda b, t: (0, 0)),
        ],
        out_shape=[
            jax.ShapeDtypeStruct((B, 1), jnp.float32),
            jax.ShapeDtypeStruct((1, 1), jnp.float32),
        ],
        interpret=interpret,
    )(pred, pred_t, gt_t)

    dist2_mean = emd_sums / float(N * D) / pcd_radius
    emd_loss = jnp.mean(dist2_mean) * 100.0
    uniform_loss = rep_sum[0, 0] / float(B * N * (NN_SIZE - 1))
    return (emd_loss, ALPHA * uniform_loss)


# speedup vs baseline: 21.8886x; 1.4509x over previous
import functools

import jax
import jax.numpy as jnp
from jax.experimental import pallas as pl

ALPHA = 1.0
NN_SIZE = 5
RADIUS = 0.07
H = 0.03
EPS = 1e-12

B = 4
N = 2048
D = 3
TILE = 512


def _loss_kernel(paug_ref, rgt_ref, rpp_ref, emd_ref, rep_ref):
    b = pl.program_id(0)
    t = pl.program_id(1)

    pa = paug_ref[0]
    rgt = rgt_ref[0]
    rpp = rpp_ref[0]

    a2 = (pa[:, 0:1] * pa[:, 0:1] + pa[:, 1:2] * pa[:, 1:2]
          + pa[:, 2:3] * pa[:, 2:3])

    s_gt = jax.lax.dot_general(pa, rgt, (((1,), (0,)), ((), ())),
                               preferred_element_type=jnp.float32,
                               precision=jax.lax.Precision.HIGHEST)
    m = jnp.min(s_gt, axis=1, keepdims=True)
    emd_val = jnp.sum(m + a2)

    s = jax.lax.dot_general(pa, rpp, (((1,), (0,)), ((), ())),
                            preferred_element_type=jnp.float32,
                            precision=jax.lax.Precision.HIGHEST)
    prev = jnp.min(s, axis=1, keepdims=True)
    rep_val = jnp.zeros((), dtype=jnp.float32)
    for _ in range(NN_SIZE - 1):
        cur = jnp.min(jnp.where(s > prev, s, jnp.inf), axis=1, keepdims=True)
        d2 = jnp.maximum(cur + a2, EPS)
        dist = jnp.sqrt(d2)
        w = jnp.exp(-d2 / (H * H))
        term = jnp.where(jnp.isfinite(cur), (RADIUS - dist) * w, 0.0)
        rep_val = rep_val + jnp.sum(term)
        prev = cur

    @pl.when(jnp.logical_and(b == 0, t == 0))
    def _():
        emd_ref[...] = jnp.zeros_like(emd_ref)
        rep_ref[...] = jnp.zeros_like(rep_ref)

    emd_ref[pl.ds(b, 1), :] += emd_val.reshape(1, 1)
    rep_ref[...] += rep_val.reshape(1, 1)


@functools.partial(jax.jit, static_argnames=("interpret",))
def kernel(pred, gt, pcd_radius, interpret=False):
    ones = jnp.ones(pred.shape[:2] + (1,), dtype=pred.dtype)
    p_aug = jnp.concatenate([pred, ones], axis=2)
    g2 = jnp.sum(gt * gt, axis=2, keepdims=True)
    rhs_gt = jnp.concatenate([-2.0 * gt, g2], axis=2).transpose(0, 2, 1)
    p2 = jnp.sum(pred * pred, axis=2, keepdims=True)
    rhs_pp = jnp.concatenate([-2.0 * pred, p2], axis=2).transpose(0, 2, 1)

    emd_sums, rep_sum = pl.pallas_call(
        _loss_kernel,
        grid=(B, N // TILE),
        in_specs=[
            pl.BlockSpec((1, TILE, 4), lambda b, t: (b, t, 0)),
            pl.BlockSpec((1, 4, N), lambda b, t: (b, 0, 0)),
            pl.BlockSpec((1, 4, N), lambda b, t: (b, 0, 0)),
        ],
        out_specs=[
            pl.BlockSpec((B, 1), lambda b, t: (0, 0)),
            pl.BlockSpec((1, 1), lambda b, t: (0, 0)),
        ],
        out_shape=[
            jax.ShapeDtypeStruct((B, 1), jnp.float32),
            jax.ShapeDtypeStruct((1, 1), jnp.float32),
        ],
        interpret=interpret,
    )(p_aug, rhs_gt, rhs_pp)

    dist2_mean = emd_sums / float(N * D) / pcd_radius
    emd_loss = jnp.mean(dist2_mean) * 100.0
    uniform_loss = rep_sum[0, 0] / float(B * N * (NN_SIZE - 1))
    return (emd_loss, ALPHA * uniform_loss)
